# Initial kernel scaffold; baseline (speedup 1.0000x reference)
import jax
import jax.numpy as jnp
from jax.experimental import pallas as pl


def kernel(input_ids, table):
    raise NotImplementedError("write your pallas kernel here")


# speedup vs baseline: 2.5300x; 2.5300x over previous
import functools

import jax
import jax.numpy as jnp
from jax import lax
from jax.experimental import pallas as pl
from jax.experimental.pallas import tpu as pltpu
from jax.experimental.pallas import tpu_sc as plsc

_B = 16384
_L = 50
_D = 32
_NC = 2
_NS = 16
_NW = _NC * _NS
_BPW = _B // _NW
_C = 16
_CHUNKS = _BPW // _C
_CI = _C * _L
_G = 80
_NG = _CI // _G
_INV = 1.0 / _L


def _cbow_sc(idx_flat, table):
    mesh = plsc.VectorSubcoreMesh(core_axis_name="c", subcore_axis_name="s")

    @functools.partial(
        pl.kernel,
        out_type=jax.ShapeDtypeStruct((_B, _D), jnp.float32),
        mesh=mesh,
        scratch_types=[
            pltpu.VMEM((_CI,), jnp.int32),
            pltpu.VMEM((_CI, _D), jnp.float32),
            pltpu.VMEM((_C, _D), jnp.float32),
            pltpu.SemaphoreType.DMA,
        ],
        compiler_params=pltpu.CompilerParams(use_tc_tiling_on_sc=False),
    )
    def k(table_hbm, idx_hbm, out_hbm, idx_v, rows_v, out_v, sem):
        wid = lax.axis_index("s") * _NC + lax.axis_index("c")
        base = wid * _BPW

        @pl.loop(jnp.int32(0), jnp.int32(_CHUNKS))
        def _chunk(c):
            row0 = base + c * _C
            pltpu.sync_copy(idx_hbm.at[pl.ds(row0 * _L, _CI)], idx_v)
            copies = [
                pltpu.async_copy(
                    table_hbm.at[idx_v.at[pl.ds(j * _G, _G)]],
                    rows_v.at[pl.ds(j * _G, _G)],
                    sem,
                )
                for j in range(_NG)
            ]
            for cp in copies:
                cp.wait()

            @pl.loop(jnp.int32(0), jnp.int32(_C))
            def _row(b):
                s = b * _L

                def body(l, accs):
                    a0, a1 = accs
                    r = s + l
                    return (a0 + rows_v[r, pl.ds(0, 16)],
                            a1 + rows_v[r, pl.ds(16, 16)])

                z = jnp.zeros((16,), jnp.float32)
                a0, a1 = lax.fori_loop(jnp.int32(0), jnp.int32(_L), body, (z, z))
                out_v[b, pl.ds(0, 16)] = a0 * _INV
                out_v[b, pl.ds(16, 16)] = a1 * _INV

            pltpu.sync_copy(out_v, out_hbm.at[pl.ds(row0, _C)])

    return k(table, idx_flat)


def kernel(input_ids, table):
    idx_flat = input_ids.reshape(-1).astype(jnp.int32)
    return _cbow_sc(idx_flat, table)
